# Initial kernel scaffold; baseline (speedup 1.0000x reference)
import jax
import jax.numpy as jnp
from jax.experimental import pallas as pl


def kernel(hidden_states, gamma, beta):
    raise NotImplementedError("write your pallas kernel here")


# speedup vs baseline: 1.8026x; 1.8026x over previous
import jax
import jax.numpy as jnp
from jax.experimental import pallas as pl

_KEEP_RATE = 0.8
_EPS = 1e-5
_BS = 512


def _scores_body(x_ref, s_ref):
    x = x_ref[...]
    s_ref[...] = jnp.sqrt(jnp.sum(x * x, axis=-1))[:, None, :]


def _mask_body(s_ref, m_ref, *, keep_k, seq):
    scores = s_ref[...]
    bits = jax.lax.bitcast_convert_type(scores, jnp.int32)
    batch = scores.shape[0]
    t = jnp.zeros((batch, 1), jnp.int32)
    for b in range(30, -1, -1):
        cand = t | (1 << b)
        cnt = jnp.sum((bits >= cand).astype(jnp.int32), axis=1, keepdims=True)
        t = jnp.where(cnt >= keep_k, cand, t)
    n_gt = jnp.sum((bits > t).astype(jnp.int32), axis=1, keepdims=True)
    eq = bits == t
    p = eq.astype(jnp.int32)
    sh = 1
    while sh < seq:
        shifted = jnp.concatenate(
            [jnp.zeros((batch, sh), jnp.int32), p[:, :-sh]], axis=1)
        p = p + shifted
        sh *= 2
    keep = (bits > t) | (eq & (p <= (keep_k - n_gt)))
    m_ref[...] = keep.astype(jnp.float32)


def _ln_body(x_ref, m_ref, g_ref, b_ref, o_ref):
    x = x_ref[0]
    m = m_ref[...]
    masked = x * m
    mu = jnp.mean(masked, axis=-1, keepdims=True)
    var = jnp.mean((masked - mu) ** 2, axis=-1, keepdims=True)
    xhat = (masked - mu) / jnp.sqrt(var + _EPS)
    o_ref[0] = xhat * g_ref[...] + b_ref[...]


def kernel(hidden_states, gamma, beta):
    batch, seq, dim = hidden_states.shape
    keep_k = max(1, int(seq * _KEEP_RATE))
    bs = min(_BS, seq)
    nblk = (batch * seq) // bs
    x3 = hidden_states.reshape(nblk, bs, dim)

    scores = pl.pallas_call(
        _scores_body,
        grid=(nblk,),
        in_specs=[pl.BlockSpec((1, bs, dim), lambda i: (i, 0, 0))],
        out_specs=pl.BlockSpec((1, 1, bs), lambda i: (i, 0, 0)),
        out_shape=jax.ShapeDtypeStruct((nblk, 1, bs), jnp.float32),
    )(x3)
    scores2 = scores.reshape(batch, seq)

    import functools
    mask = pl.pallas_call(
        functools.partial(_mask_body, keep_k=keep_k, seq=seq),
        in_specs=[pl.BlockSpec((batch, seq), lambda: (0, 0))],
        out_specs=pl.BlockSpec((batch, seq), lambda: (0, 0)),
        out_shape=jax.ShapeDtypeStruct((batch, seq), jnp.float32),
    )(scores2)

    mask_col = mask.reshape(batch * seq, 1)
    out = pl.pallas_call(
        _ln_body,
        grid=(nblk,),
        in_specs=[
            pl.BlockSpec((1, bs, dim), lambda i: (i, 0, 0)),
            pl.BlockSpec((bs, 1), lambda i: (i, 0)),
            pl.BlockSpec((dim,), lambda i: (0,)),
            pl.BlockSpec((dim,), lambda i: (0,)),
        ],
        out_specs=pl.BlockSpec((1, bs, dim), lambda i: (i, 0, 0)),
        out_shape=jax.ShapeDtypeStruct((nblk, bs, dim), jnp.float32),
    )(x3, mask_col, gamma, beta)
    return out.reshape(batch, seq, dim)
